# baseline (speedup 1.0000x reference)
import functools

import jax
import jax.numpy as jnp
from jax import lax
from jax.experimental import pallas as pl
from jax.experimental.pallas import tpu as pltpu
from jax.experimental.pallas import tpu_sc as plsc

VOCAB = 1000
WMAIN = 896
WTAIL = VOCAB - WMAIN
N_EMBD = 32
B, T = 1024, 200
BT = B * T

NC, NS = 2, 16
NW = NC * NS
B_PER_W = BT // NW
BATCH_PER_W = B // NW
CH = 40
CPB = T // CH
N_CH = B_PER_W // CH
NP = N_CH // 2

_TAIL_OFFS = (0, 16, 32, 48, 64, 80)


def _table_body(emb_ref, wa_ref, wb_ref, ba_ref, bb_ref, outa_ref, outb_ref):
    e = emb_ref[...]
    outa_ref[...] = (
        jnp.dot(e, wa_ref[...], preferred_element_type=jnp.float32)
        + ba_ref[...]
    )
    outb_ref[...] = (
        jnp.dot(e, wb_ref[...], preferred_element_type=jnp.float32)
        + bb_ref[...]
    )


def _fused_tables(emb_table, W, b):
    wa = W[:, :WMAIN]
    wb = jnp.pad(W[:, WMAIN:], ((0, 0), (0, 128 - WTAIL)))
    ba = b[:WMAIN].reshape(1, WMAIN)
    bb = jnp.pad(b[WMAIN:], (0, 128 - WTAIL)).reshape(1, 128)
    return pl.pallas_call(
        _table_body,
        out_shape=(
            jax.ShapeDtypeStruct((VOCAB, WMAIN), jnp.float32),
            jax.ShapeDtypeStruct((VOCAB, 128), jnp.float32),
        ),
    )(emb_table, wa, wb, ba, bb)


_mesh = plsc.VectorSubcoreMesh(core_axis_name="c", subcore_axis_name="s")


@functools.partial(
    pl.kernel,
    mesh=_mesh,
    out_type=jax.ShapeDtypeStruct((B, T, VOCAB), jnp.float32),
    scratch_types=[
        pltpu.VMEM((B_PER_W,), jnp.int32),
        pltpu.VMEM((CH, VOCAB), jnp.float32),
        pltpu.VMEM((CH, VOCAB), jnp.float32),
        pltpu.VMEM((CH, 128), jnp.float32),
        pltpu.VMEM((CH, 128), jnp.float32),
        pltpu.SemaphoreType.DMA,
        pltpu.SemaphoreType.DMA,
        pltpu.SemaphoreType.DMA,
        pltpu.SemaphoreType.DMA,
    ],
    compiler_params=pltpu.CompilerParams(
        use_tc_tiling_on_sc=True, needs_layout_passes=False
    ),
)
def _sc_gather(tbla_hbm, tblb_hbm, idx_hbm, out_hbm, idx_v, rows_a, rows_b,
               tail_a, tail_b, gsem_a, gsem_b, ssem_a, ssem_b):
    c = lax.axis_index("c")
    s = lax.axis_index("s")
    wid = s * NC + c
    base = wid * B_PER_W
    base_b = wid * BATCH_PER_W

    pltpu.sync_copy(idx_hbm.at[pl.ds(base, B_PER_W)], idx_v)

    def start_gather(g, rows, tail, sem):
        idxs = idx_v.at[pl.ds(g * CH, CH)]
        pltpu.async_copy(tbla_hbm.at[idxs], rows.at[:, pl.ds(0, WMAIN)], sem)
        pltpu.async_copy(tblb_hbm.at[idxs], tail, sem)

    def wait_gather(rows, tail, sem):
        idxs = idx_v.at[pl.ds(0, CH)]
        pltpu.make_async_copy(
            tbla_hbm.at[idxs], rows.at[:, pl.ds(0, WMAIN)], sem
        ).wait()
        pltpu.make_async_copy(tblb_hbm.at[idxs], tail, sem).wait()

    def fix_tail(rows, tail):
        lanes = lax.iota(jnp.int32, 16)
        rem_mask = lanes < (WTAIL - 96)
        for r in range(CH):
            for off in _TAIL_OFFS:
                rows[r, pl.ds(WMAIN + off, 16)] = tail[r, pl.ds(off, 16)]
            x = tail[r, pl.ds(96, 16)]
            plsc.store_scatter(
                rows,
                [jnp.full((16,), r, jnp.int32), WMAIN + 96 + lanes],
                x,
                mask=rem_mask,
            )

    def start_scatter(g, rows, sem):
        bi = base_b + g // CPB
        t0 = (g % CPB) * CH
        pltpu.async_copy(rows, out_hbm.at[bi, pl.ds(t0, CH)], sem)

    def wait_scatter(rows, sem):
        pltpu.make_async_copy(
            rows, out_hbm.at[base_b, pl.ds(0, CH)], sem
        ).wait()

    start_gather(0, rows_a, tail_a, gsem_a)
    start_gather(1, rows_b, tail_b, gsem_b)

    def body(p, carry):
        g0 = 2 * p
        wait_gather(rows_a, tail_a, gsem_a)
        fix_tail(rows_a, tail_a)
        start_scatter(g0, rows_a, ssem_a)
        wait_gather(rows_b, tail_b, gsem_b)
        fix_tail(rows_b, tail_b)
        start_scatter(g0 + 1, rows_b, ssem_b)
        wait_scatter(rows_a, ssem_a)
        start_gather(jnp.minimum(g0 + 2, N_CH - 2), rows_a, tail_a, gsem_a)
        wait_scatter(rows_b, ssem_b)
        start_gather(jnp.minimum(g0 + 3, N_CH - 1), rows_b, tail_b, gsem_b)
        return carry

    lax.fori_loop(0, NP, body, 0)
    wait_gather(rows_a, tail_a, gsem_a)
    wait_gather(rows_b, tail_b, gsem_b)


def kernel(idx, emb_table, W, b):
    tbla, tblb = _fused_tables(emb_table, W, b)
    flat_idx = idx.reshape(-1).astype(jnp.int32)
    return _sc_gather(tbla, tblb, flat_idx)


# speedup vs baseline: 2.1419x; 2.1419x over previous
import functools

import jax
import jax.numpy as jnp
from jax import lax
from jax.experimental import pallas as pl
from jax.experimental.pallas import tpu as pltpu
from jax.experimental.pallas import tpu_sc as plsc
from jax.experimental import layout as jex_layout

VOCAB = 1000
WMAIN = 896
WTAIL = VOCAB - WMAIN
N_EMBD = 32
B, T = 1024, 200
BT = B * T

NC, NS = 2, 16
NW = NC * NS
B_PER_W = BT // NW
BATCH_PER_W = B // NW
CH = 40
CPB = T // CH
N_CH = B_PER_W // CH
NP = N_CH // 2

_TAIL_OFFS = (0, 16, 32, 48, 64, 80)


def _table_body(emb_ref, wa_ref, wb_ref, ba_ref, bb_ref, outa_ref, outb_ref):
    e = emb_ref[...]
    outa_ref[...] = (
        jnp.dot(e, wa_ref[...], preferred_element_type=jnp.float32)
        + ba_ref[...]
    )
    outb_ref[...] = (
        jnp.dot(e, wb_ref[...], preferred_element_type=jnp.float32)
        + bb_ref[...]
    )


def _fused_tables(emb_table, W, b):
    wa = W[:, :WMAIN]
    wb = jnp.pad(W[:, WMAIN:], ((0, 0), (0, 128 - WTAIL)))
    ba = b[:WMAIN].reshape(1, WMAIN)
    bb = jnp.pad(b[WMAIN:], (0, 128 - WTAIL)).reshape(1, 128)
    return pl.pallas_call(
        _table_body,
        out_shape=(
            jax.ShapeDtypeStruct((VOCAB, WMAIN), jnp.float32),
            jax.ShapeDtypeStruct((VOCAB, 128), jnp.float32),
        ),
    )(emb_table, wa, wb, ba, bb)


_mesh = plsc.VectorSubcoreMesh(core_axis_name="c", subcore_axis_name="s")


@functools.partial(
    pl.kernel,
    mesh=_mesh,
    out_type=jax.ShapeDtypeStruct((B, T, VOCAB), jnp.float32),
    scratch_types=[
        pltpu.VMEM((B_PER_W,), jnp.int32),
        pltpu.VMEM((CH, VOCAB), jnp.float32),
        pltpu.VMEM((CH, VOCAB), jnp.float32),
        pltpu.VMEM((CH, 128), jnp.float32),
        pltpu.VMEM((CH, 128), jnp.float32),
        pltpu.SemaphoreType.DMA,
        pltpu.SemaphoreType.DMA,
        pltpu.SemaphoreType.DMA,
        pltpu.SemaphoreType.DMA,
    ],
    compiler_params=pltpu.CompilerParams(
        use_tc_tiling_on_sc=True, needs_layout_passes=False
    ),
)
def _sc_gather(tbla_hbm, tblb_hbm, idx_hbm, out_hbm, idx_v, rows_a, rows_b,
               tail_a, tail_b, gsem_a, gsem_b, ssem_a, ssem_b):
    c = lax.axis_index("c")
    s = lax.axis_index("s")
    wid = s * NC + c
    base = wid * B_PER_W
    base_b = wid * BATCH_PER_W

    pltpu.sync_copy(idx_hbm.at[pl.ds(base, B_PER_W)], idx_v)

    def start_gather(g, rows, tail, sem):
        idxs = idx_v.at[pl.ds(g * CH, CH)]
        pltpu.async_copy(tbla_hbm.at[idxs], rows.at[:, pl.ds(0, WMAIN)], sem)
        pltpu.async_copy(tblb_hbm.at[idxs], tail, sem)

    def wait_gather(rows, tail, sem):
        idxs = idx_v.at[pl.ds(0, CH)]
        pltpu.make_async_copy(
            tbla_hbm.at[idxs], rows.at[:, pl.ds(0, WMAIN)], sem
        ).wait()
        pltpu.make_async_copy(tblb_hbm.at[idxs], tail, sem).wait()

    def fix_tail(rows, tail):
        lanes = lax.iota(jnp.int32, 16)
        rem_mask = lanes < (WTAIL - 96)
        for r in range(CH):
            for off in _TAIL_OFFS:
                rows[r, pl.ds(WMAIN + off, 16)] = tail[r, pl.ds(off, 16)]
            x = tail[r, pl.ds(96, 16)]
            plsc.store_scatter(
                rows,
                [jnp.full((16,), r, jnp.int32), WMAIN + 96 + lanes],
                x,
                mask=rem_mask,
            )

    def start_scatter(g, rows, sem):
        bi = base_b + g // CPB
        t0 = (g % CPB) * CH
        pltpu.async_copy(rows, out_hbm.at[bi, pl.ds(t0, CH)], sem)

    def wait_scatter(rows, sem):
        pltpu.make_async_copy(
            rows, out_hbm.at[base_b, pl.ds(0, CH)], sem
        ).wait()

    start_gather(0, rows_a, tail_a, gsem_a)
    start_gather(1, rows_b, tail_b, gsem_b)

    def body(p, carry):
        g0 = 2 * p
        wait_gather(rows_a, tail_a, gsem_a)
        fix_tail(rows_a, tail_a)
        start_scatter(g0, rows_a, ssem_a)
        wait_gather(rows_b, tail_b, gsem_b)
        fix_tail(rows_b, tail_b)
        start_scatter(g0 + 1, rows_b, ssem_b)
        wait_scatter(rows_a, ssem_a)
        start_gather(jnp.minimum(g0 + 2, N_CH - 2), rows_a, tail_a, gsem_a)
        wait_scatter(rows_b, ssem_b)
        start_gather(jnp.minimum(g0 + 3, N_CH - 1), rows_b, tail_b, gsem_b)
        return carry

    lax.fori_loop(0, NP, body, 0)
    wait_gather(rows_a, tail_a, gsem_a)
    wait_gather(rows_b, tail_b, gsem_b)


def kernel(idx, emb_table, W, b):
    tbla, tblb = _fused_tables(emb_table, W, b)
    flat_idx = idx.reshape(-1).astype(jnp.int32)
    out = _sc_gather(tbla, tblb, flat_idx)
    return jex_layout.with_layout_constraint(
        out, jex_layout.Layout(major_to_minor=(0, 1, 2))
    )
